# baseline (speedup 1.0000x reference)
import jax
import jax.numpy as jnp
from jax import lax
from jax.experimental import pallas as pl
from jax.experimental.pallas import tpu as pltpu
from jax.experimental.pallas import tpu_sc as plsc

N = 10000
E = 320000
D_IN, D_HID, D_OUT = 128, 256, 128
NC, NS = 2, 16
CHUNK = 128
NP = 10240
RPT = NP // NS
K2 = 80
GSZ = 20
K0 = 80
K1 = 80
NCH = NS * (K0 + K1)
EP = NCH * CHUNK

_MESH = plsc.VectorSubcoreMesh(core_axis_name="c", subcore_axis_name="s")
_f32 = jnp.float32


def _deg_body(dst_hbm, ones_hbm, degcat, idx_v, ones_v, acc):
    c = lax.axis_index("c")
    s = lax.axis_index("s")
    w = s * NC + c
    pltpu.sync_copy(dst_hbm.at[w], idx_v)
    pltpu.sync_copy(ones_hbm.at[pl.ds(0, CHUNK)], ones_v)
    pltpu.sync_copy(ones_hbm.at[pl.ds(s * RPT, RPT)], acc.at[pl.ds(s * RPT, RPT)])
    plsc.subcore_barrier()

    def body(j, carry):
        pltpu.sync_copy(ones_v, acc.at[idx_v.at[j]], add=True)
        return carry

    lax.fori_loop(0, K2, body, 0)
    plsc.subcore_barrier()
    pltpu.sync_copy(acc.at[pl.ds(s * RPT, RPT)],
                    degcat.at[pl.ds(c * NP + s * RPT, RPT)])


_deg_kernel = pl.kernel(
    _deg_body,
    out_type=jax.ShapeDtypeStruct((NC * NP, 16), _f32),
    mesh=_MESH,
    scratch_types=[
        pltpu.VMEM((K2, CHUNK), jnp.int32),
        pltpu.VMEM((CHUNK, 16), _f32),
        pltpu.VMEM_SHARED((NP, 16), _f32),
    ],
)


def _agg_body(src_hbm, dst_hbm, table_hbm, out_hbm,
              idx_sA, idx_dA, idx_sB, idx_dB, b0, b1,
              semA, semB, psemA, psemB, acc):
    c = lax.axis_index("c")
    s = lax.axis_index("s")
    pltpu.sync_copy(table_hbm.at[pl.ds(s * RPT, RPT)],
                    acc.at[pl.ds(s * RPT, RPT)])
    plsc.subcore_barrier()

    dummy = table_hbm.at[pl.ds(0, CHUNK)]
    dummy_i = src_hbm.at[0]

    def gather(idx_row, buf, sem):
        pltpu.async_copy(table_hbm.at[idx_row], buf, sem)

    def prefetch(g, idx_s, idx_d, psem):
        pltpu.async_copy(src_hbm.at[g], idx_s, psem)
        pltpu.async_copy(dst_hbm.at[g], idx_d, psem)

    def process(idx_s, idx_d):
        gather(idx_s.at[0], b0, semA)
        gather(idx_s.at[1], b1, semB)

        def pair(t, carry2):
            pltpu.make_async_copy(dummy, b0, semA).wait()
            pltpu.sync_copy(b0, acc.at[idx_d.at[2 * t]], add=True)

            @pl.when(2 * t + 2 < GSZ)
            def _():
                gather(idx_s.at[2 * t + 2], b0, semA)

            pltpu.make_async_copy(dummy, b1, semB).wait()
            pltpu.sync_copy(b1, acc.at[idx_d.at[2 * t + 1]], add=True)

            @pl.when(2 * t + 3 < GSZ)
            def _():
                gather(idx_s.at[2 * t + 3], b1, semB)

            return carry2

        lax.fori_loop(0, GSZ // 2, pair, 0)

    def drain_idx(idx_s, idx_d, psem):
        pltpu.make_async_copy(dummy_i, idx_s, psem).wait()
        pltpu.make_async_copy(dummy_i, idx_d, psem).wait()

    def run(gs, ngroups):
        prefetch(gs, idx_sA, idx_dA, psemA)

        def duo(u, carry):
            g = 2 * u
            drain_idx(idx_sA, idx_dA, psemA)

            @pl.when(g + 1 < ngroups)
            def _():
                prefetch(gs + g + 1, idx_sB, idx_dB, psemB)

            process(idx_sA, idx_dA)

            @pl.when(g + 1 < ngroups)
            def _():
                drain_idx(idx_sB, idx_dB, psemB)

                @pl.when(g + 2 < ngroups)
                def _():
                    prefetch(gs + g + 2, idx_sA, idx_dA, psemA)

                process(idx_sB, idx_dB)

            return carry

        lax.fori_loop(0, (ngroups + 1) // 2, duo, 0)

    @pl.when(c == 0)
    def _():
        run(s * (K0 // GSZ), K0 // GSZ)

    @pl.when(c == 1)
    def _():
        run(NS * (K0 // GSZ) + s * (K1 // GSZ), K1 // GSZ)

    plsc.subcore_barrier()
    pltpu.sync_copy(acc.at[pl.ds(s * RPT, RPT)],
                    out_hbm.at[pl.ds(c * NP + s * RPT, RPT)])


_agg_kernel = pl.kernel(
    _agg_body,
    out_type=jax.ShapeDtypeStruct((NC * NP, 128), _f32),
    mesh=_MESH,
    scratch_types=[
        pltpu.VMEM((GSZ, CHUNK), jnp.int32),
        pltpu.VMEM((GSZ, CHUNK), jnp.int32),
        pltpu.VMEM((GSZ, CHUNK), jnp.int32),
        pltpu.VMEM((GSZ, CHUNK), jnp.int32),
        pltpu.VMEM((CHUNK, 128), _f32),
        pltpu.VMEM((CHUNK, 128), _f32),
        pltpu.SemaphoreType.DMA,
        pltpu.SemaphoreType.DMA,
        pltpu.SemaphoreType.DMA,
        pltpu.SemaphoreType.DMA,
        pltpu.VMEM_SHARED((NP, 128), _f32),
    ],
)


_BR = 1024
_NB = NP // _BR


def _tc1_body(x_ref, w1_ref, dga_ref, dgb_ref, ya_ref, yb_ref, dis_ref):
    deg = dga_ref[:, 0:1] + dgb_ref[:, 0:1] - 1.0
    dis = lax.rsqrt(deg)
    y = jnp.dot(x_ref[...], w1_ref[...], preferred_element_type=_f32) * dis
    ya_ref[...] = y[:, 0:128]
    yb_ref[...] = y[:, 128:256]
    dis_ref[...] = dis


def _tc1(xp, W1, degcat):
    return pl.pallas_call(
        _tc1_body,
        grid=(_NB,),
        in_specs=[
            pl.BlockSpec((_BR, D_IN), lambda i: (i, 0)),
            pl.BlockSpec((D_IN, D_HID), lambda i: (0, 0)),
            pl.BlockSpec((_BR, 16), lambda i: (i, 0)),
            pl.BlockSpec((_BR, 16), lambda i: (_NB + i, 0)),
        ],
        out_specs=[
            pl.BlockSpec((_BR, 128), lambda i: (i, 0)),
            pl.BlockSpec((_BR, 128), lambda i: (i, 0)),
            pl.BlockSpec((_BR, 1), lambda i: (i, 0)),
        ],
        out_shape=[
            jax.ShapeDtypeStruct((NP, 128), _f32),
            jax.ShapeDtypeStruct((NP, 128), _f32),
            jax.ShapeDtypeStruct((NP, 1), _f32),
        ],
    )(xp, W1, degcat, degcat)


def _tc2_body(pa0, pa1, pb0, pb1, ya_ref, yb_ref, dis_ref, b1_ref, w2_ref,
              y2_ref):
    dis = dis_ref[...]
    agga = pa0[...] + pa1[...] - ya_ref[...]
    aggb = pb0[...] + pb1[...] - yb_ref[...]
    ha = jnp.maximum(dis * agga + b1_ref[0:1, 0:128], 0.0)
    hb = jnp.maximum(dis * aggb + b1_ref[0:1, 128:256], 0.0)
    y2 = (jnp.dot(ha, w2_ref[0:128, :], preferred_element_type=_f32)
          + jnp.dot(hb, w2_ref[128:256, :], preferred_element_type=_f32))
    y2_ref[...] = y2 * dis


def _tc2(pa, pb, ya, yb, dis, b1r, W2):
    lo = pl.BlockSpec((_BR, 128), lambda i: (i, 0))
    hi = pl.BlockSpec((_BR, 128), lambda i: (_NB + i, 0))
    return pl.pallas_call(
        _tc2_body,
        grid=(_NB,),
        in_specs=[lo, hi, lo, hi, lo, lo,
                  pl.BlockSpec((_BR, 1), lambda i: (i, 0)),
                  pl.BlockSpec((1, D_HID), lambda i: (0, 0)),
                  pl.BlockSpec((D_HID, D_OUT), lambda i: (0, 0))],
        out_specs=pl.BlockSpec((_BR, D_OUT), lambda i: (i, 0)),
        out_shape=jax.ShapeDtypeStruct((NP, D_OUT), _f32),
    )(pa, pa, pb, pb, ya, yb, dis, b1r, W2)


def _tc3_body(p0_ref, p1_ref, y2_ref, dis_ref, b2_ref, o_ref):
    o_ref[...] = (dis_ref[...] * (p0_ref[...] + p1_ref[...] - y2_ref[...])
                  + b2_ref[...])


def _tc3(pc, y2, dis, b2r):
    return pl.pallas_call(
        _tc3_body,
        grid=(_NB,),
        in_specs=[
            pl.BlockSpec((_BR, D_OUT), lambda i: (i, 0)),
            pl.BlockSpec((_BR, D_OUT), lambda i: (_NB + i, 0)),
            pl.BlockSpec((_BR, D_OUT), lambda i: (i, 0)),
            pl.BlockSpec((_BR, 1), lambda i: (i, 0)),
            pl.BlockSpec((1, D_OUT), lambda i: (0, 0)),
        ],
        out_specs=pl.BlockSpec((_BR, D_OUT), lambda i: (i, 0)),
        out_shape=jax.ShapeDtypeStruct((NP, D_OUT), _f32),
    )(pc, pc, y2, dis, b2r)


def kernel(x, edge_index, W1, b1, W2, b2):
    src = edge_index[0].astype(jnp.int32)
    dst = edge_index[1].astype(jnp.int32)
    pad = EP - E
    padrows = N + jnp.arange(pad, dtype=jnp.int32) % (NP - N)
    src = jnp.concatenate([src, padrows])
    dst = jnp.concatenate([dst, padrows])
    srcB = src.reshape(NCH // GSZ, GSZ, CHUNK)
    dstB = dst.reshape(NCH // GSZ, GSZ, CHUNK)
    dstB128 = dst.reshape(NC * NS, K2, CHUNK)
    ones16 = jnp.ones((NP, 16), _f32)
    b1r = b1.reshape(1, D_HID)
    b2r = b2.reshape(1, D_OUT)
    xp = jnp.concatenate([x, jnp.zeros((NP - N, D_IN), _f32)])

    degcat = _deg_kernel(dstB128, ones16)
    ya, yb, dis = _tc1(xp, W1, degcat)
    pa = _agg_kernel(srcB, dstB, ya)
    pb = _agg_kernel(srcB, dstB, yb)
    y2 = _tc2(pa, pb, ya, yb, dis, b1r, W2)
    pc = _agg_kernel(srcB, dstB, y2)
    return _tc3(pc, y2, dis, b2r)[:N]


# speedup vs baseline: 1.0051x; 1.0051x over previous
import jax
import jax.numpy as jnp
from jax import lax
from jax.experimental import pallas as pl
from jax.experimental.pallas import tpu as pltpu
from jax.experimental.pallas import tpu_sc as plsc

N = 10000
E = 320000
D_IN, D_HID, D_OUT = 128, 256, 128
NC, NS = 2, 16
CHUNK = 128
NP = 10240
RPT = NP // NS
K2 = 80
GSZ = 20
K0 = 80
K1 = 80
NCH = NS * (K0 + K1)
EP = NCH * CHUNK

_MESH = plsc.VectorSubcoreMesh(core_axis_name="c", subcore_axis_name="s")
_f32 = jnp.float32


def _deg_body(dst_hbm, ones_hbm, degcat, idx_v, ones_v, acc):
    c = lax.axis_index("c")
    s = lax.axis_index("s")
    w = s * NC + c
    pltpu.sync_copy(dst_hbm.at[w], idx_v)
    pltpu.sync_copy(ones_hbm.at[pl.ds(0, CHUNK)], ones_v)
    pltpu.sync_copy(ones_hbm.at[pl.ds(s * RPT, RPT)], acc.at[pl.ds(s * RPT, RPT)])
    plsc.subcore_barrier()

    def body(j, carry):
        pltpu.sync_copy(ones_v, acc.at[idx_v.at[j]], add=True)
        return carry

    lax.fori_loop(0, K2, body, 0)
    plsc.subcore_barrier()
    pltpu.sync_copy(acc.at[pl.ds(s * RPT, RPT)],
                    degcat.at[pl.ds(c * NP + s * RPT, RPT)])


_deg_kernel = pl.kernel(
    _deg_body,
    out_type=jax.ShapeDtypeStruct((NC * NP, 16), _f32),
    mesh=_MESH,
    scratch_types=[
        pltpu.VMEM((K2, CHUNK), jnp.int32),
        pltpu.VMEM((CHUNK, 16), _f32),
        pltpu.VMEM_SHARED((NP, 16), _f32),
    ],
)


def _agg_body(src_hbm, dst_hbm, table_hbm, zero_hbm, out_hbm,
              idx_sA, idx_dA, idx_sB, idx_dB, b0, b1,
              semA, semB, psemA, psemB, acc):
    c = lax.axis_index("c")
    s = lax.axis_index("s")
    @pl.when(c == 0)
    def _():
        pltpu.sync_copy(table_hbm.at[pl.ds(s * RPT, RPT)],
                        acc.at[pl.ds(s * RPT, RPT)])

    @pl.when(c == 1)
    def _():
        pltpu.sync_copy(zero_hbm.at[pl.ds(s * RPT, RPT)],
                        acc.at[pl.ds(s * RPT, RPT)])

    plsc.subcore_barrier()

    dummy = table_hbm.at[pl.ds(0, CHUNK)]
    dummy_i = src_hbm.at[0]

    def gather(idx_row, buf, sem):
        pltpu.async_copy(table_hbm.at[idx_row], buf, sem)

    def prefetch(g, idx_s, idx_d, psem):
        pltpu.async_copy(src_hbm.at[g], idx_s, psem)
        pltpu.async_copy(dst_hbm.at[g], idx_d, psem)

    def process(idx_s, idx_d):
        gather(idx_s.at[0], b0, semA)
        gather(idx_s.at[1], b1, semB)

        def pair(t, carry2):
            pltpu.make_async_copy(dummy, b0, semA).wait()
            pltpu.sync_copy(b0, acc.at[idx_d.at[2 * t]], add=True)

            @pl.when(2 * t + 2 < GSZ)
            def _():
                gather(idx_s.at[2 * t + 2], b0, semA)

            pltpu.make_async_copy(dummy, b1, semB).wait()
            pltpu.sync_copy(b1, acc.at[idx_d.at[2 * t + 1]], add=True)

            @pl.when(2 * t + 3 < GSZ)
            def _():
                gather(idx_s.at[2 * t + 3], b1, semB)

            return carry2

        lax.fori_loop(0, GSZ // 2, pair, 0)

    def drain_idx(idx_s, idx_d, psem):
        pltpu.make_async_copy(dummy_i, idx_s, psem).wait()
        pltpu.make_async_copy(dummy_i, idx_d, psem).wait()

    def run(gs, ngroups):
        prefetch(gs, idx_sA, idx_dA, psemA)

        def duo(u, carry):
            g = 2 * u
            drain_idx(idx_sA, idx_dA, psemA)

            @pl.when(g + 1 < ngroups)
            def _():
                prefetch(gs + g + 1, idx_sB, idx_dB, psemB)

            process(idx_sA, idx_dA)

            @pl.when(g + 1 < ngroups)
            def _():
                drain_idx(idx_sB, idx_dB, psemB)

                @pl.when(g + 2 < ngroups)
                def _():
                    prefetch(gs + g + 2, idx_sA, idx_dA, psemA)

                process(idx_sB, idx_dB)

            return carry

        lax.fori_loop(0, (ngroups + 1) // 2, duo, 0)

    @pl.when(c == 0)
    def _():
        run(s * (K0 // GSZ), K0 // GSZ)

    @pl.when(c == 1)
    def _():
        run(NS * (K0 // GSZ) + s * (K1 // GSZ), K1 // GSZ)

    plsc.subcore_barrier()
    pltpu.sync_copy(acc.at[pl.ds(s * RPT, RPT)],
                    out_hbm.at[pl.ds(c * NP + s * RPT, RPT)])


_agg_kernel = pl.kernel(
    _agg_body,
    out_type=jax.ShapeDtypeStruct((NC * NP, 128), _f32),
    mesh=_MESH,
    scratch_types=[
        pltpu.VMEM((GSZ, CHUNK), jnp.int32),
        pltpu.VMEM((GSZ, CHUNK), jnp.int32),
        pltpu.VMEM((GSZ, CHUNK), jnp.int32),
        pltpu.VMEM((GSZ, CHUNK), jnp.int32),
        pltpu.VMEM((CHUNK, 128), _f32),
        pltpu.VMEM((CHUNK, 128), _f32),
        pltpu.SemaphoreType.DMA,
        pltpu.SemaphoreType.DMA,
        pltpu.SemaphoreType.DMA,
        pltpu.SemaphoreType.DMA,
        pltpu.VMEM_SHARED((NP, 128), _f32),
    ],
)


_BR = 1024
_NB = NP // _BR


def _tc1_body(x_ref, w1_ref, dga_ref, dgb_ref, ya_ref, yb_ref, dis_ref):
    deg = dga_ref[:, 0:1] + dgb_ref[:, 0:1] - 1.0
    dis = lax.rsqrt(deg)
    y = jnp.dot(x_ref[...], w1_ref[...], preferred_element_type=_f32) * dis
    ya_ref[...] = y[:, 0:128]
    yb_ref[...] = y[:, 128:256]
    dis_ref[...] = dis


def _tc1(xp, W1, degcat):
    return pl.pallas_call(
        _tc1_body,
        grid=(_NB,),
        in_specs=[
            pl.BlockSpec((_BR, D_IN), lambda i: (i, 0)),
            pl.BlockSpec((D_IN, D_HID), lambda i: (0, 0)),
            pl.BlockSpec((_BR, 16), lambda i: (i, 0)),
            pl.BlockSpec((_BR, 16), lambda i: (_NB + i, 0)),
        ],
        out_specs=[
            pl.BlockSpec((_BR, 128), lambda i: (i, 0)),
            pl.BlockSpec((_BR, 128), lambda i: (i, 0)),
            pl.BlockSpec((_BR, 1), lambda i: (i, 0)),
        ],
        out_shape=[
            jax.ShapeDtypeStruct((NP, 128), _f32),
            jax.ShapeDtypeStruct((NP, 128), _f32),
            jax.ShapeDtypeStruct((NP, 1), _f32),
        ],
    )(xp, W1, degcat, degcat)


def _tc2_body(pa0, pa1, pb0, pb1, dis_ref, b1_ref, w2_ref, y2_ref):
    dis = dis_ref[...]
    agga = pa0[...] + pa1[...]
    aggb = pb0[...] + pb1[...]
    ha = jnp.maximum(dis * agga + b1_ref[0:1, 0:128], 0.0)
    hb = jnp.maximum(dis * aggb + b1_ref[0:1, 128:256], 0.0)
    y2 = (jnp.dot(ha, w2_ref[0:128, :], preferred_element_type=_f32)
          + jnp.dot(hb, w2_ref[128:256, :], preferred_element_type=_f32))
    y2_ref[...] = y2 * dis


def _tc2(pa, pb, dis, b1r, W2):
    lo = pl.BlockSpec((_BR, 128), lambda i: (i, 0))
    hi = pl.BlockSpec((_BR, 128), lambda i: (_NB + i, 0))
    return pl.pallas_call(
        _tc2_body,
        grid=(_NB,),
        in_specs=[lo, hi, lo, hi,
                  pl.BlockSpec((_BR, 1), lambda i: (i, 0)),
                  pl.BlockSpec((1, D_HID), lambda i: (0, 0)),
                  pl.BlockSpec((D_HID, D_OUT), lambda i: (0, 0))],
        out_specs=pl.BlockSpec((_BR, D_OUT), lambda i: (i, 0)),
        out_shape=jax.ShapeDtypeStruct((NP, D_OUT), _f32),
    )(pa, pa, pb, pb, dis, b1r, W2)


def _tc3_body(p0_ref, p1_ref, dis_ref, b2_ref, o_ref):
    o_ref[...] = (dis_ref[...] * (p0_ref[...] + p1_ref[...]) + b2_ref[...])


def _tc3(pc, dis, b2r):
    return pl.pallas_call(
        _tc3_body,
        grid=(_NB,),
        in_specs=[
            pl.BlockSpec((_BR, D_OUT), lambda i: (i, 0)),
            pl.BlockSpec((_BR, D_OUT), lambda i: (_NB + i, 0)),
            pl.BlockSpec((_BR, 1), lambda i: (i, 0)),
            pl.BlockSpec((1, D_OUT), lambda i: (0, 0)),
        ],
        out_specs=pl.BlockSpec((_BR, D_OUT), lambda i: (i, 0)),
        out_shape=jax.ShapeDtypeStruct((NP, D_OUT), _f32),
    )(pc, pc, dis, b2r)


def kernel(x, edge_index, W1, b1, W2, b2):
    src = edge_index[0].astype(jnp.int32)
    dst = edge_index[1].astype(jnp.int32)
    pad = EP - E
    padrows = N + jnp.arange(pad, dtype=jnp.int32) % (NP - N)
    src = jnp.concatenate([src, padrows])
    dst = jnp.concatenate([dst, padrows])
    srcB = src.reshape(NCH // GSZ, GSZ, CHUNK)
    dstB = dst.reshape(NCH // GSZ, GSZ, CHUNK)
    dstB128 = dst.reshape(NC * NS, K2, CHUNK)
    ones16 = jnp.ones((NP, 16), _f32)
    b1r = b1.reshape(1, D_HID)
    b2r = b2.reshape(1, D_OUT)
    xp = jnp.concatenate([x, jnp.zeros((NP - N, D_IN), _f32)])

    zero = jnp.zeros((NP, 128), _f32)
    degcat = _deg_kernel(dstB128, ones16)
    ya, yb, dis = _tc1(xp, W1, degcat)
    pa = _agg_kernel(srcB, dstB, ya, zero)
    pb = _agg_kernel(srcB, dstB, yb, zero)
    y2 = _tc2(pa, pb, dis, b1r, W2)
    pc = _agg_kernel(srcB, dstB, y2, zero)
    return _tc3(pc, dis, b2r)[:N]
